# baseline (speedup 1.0000x reference)
import functools

import jax
import jax.numpy as jnp
from jax.experimental import pallas as pl


def _vq_block_kernel(ze_ref, emb_ref, ecat_ref, zq_ref, idx_ref,
                     cnt_ref, loss_ref, *, num_codes):
    i = pl.program_id(0)
    ze = ze_ref[0]
    emb = emb_ref[...]
    ecat = ecat_ref[...]
    d = ze.shape[0]
    tblk = ze.shape[1]

    e_sq = jnp.sum(emb * emb, axis=1)
    x_sq = jnp.sum(ze * ze, axis=0)
    dot = jax.lax.dot_general(
        ecat[:, :d], ze.astype(jnp.bfloat16), (((1,), (0,)), ((), ())),
        preferred_element_type=jnp.float32,
    )
    dist = (x_sq[None, :] - 2.0 * dot) + e_sq[:, None]

    idx = jnp.argmin(dist, axis=0).astype(jnp.int32)
    iota = jax.lax.broadcasted_iota(jnp.int32, (num_codes, 1), 0)
    onehot = (iota == idx[None, :]).astype(jnp.bfloat16)
    zq2 = jax.lax.dot_general(
        ecat, onehot, (((0,), (0,)), ((), ())),
        preferred_element_type=jnp.float32)
    zq = zq2[:d] + zq2[d:]

    zq_ref[0] = zq
    idx_ref[0, 0, :] = idx

    ones_col = jnp.ones((1, tblk), jnp.bfloat16)
    blk_counts = jax.lax.dot_general(
        ones_col, onehot, (((1,), (1,)), ((), ())),
        preferred_element_type=jnp.float32)
    blk_loss = jnp.full((128,), jnp.sum((ze - zq) ** 2), jnp.float32)

    @pl.when(i == 0)
    def _init():
        cnt_ref[0, :] = blk_counts[0]
        loss_ref[0, :] = blk_loss

    @pl.when(i > 0)
    def _accum():
        cnt_ref[0, :] += blk_counts[0]
        loss_ref[0, :] += blk_loss


def kernel(z_e, embedding):
    B, d, T = z_e.shape
    K = embedding.shape[0]
    Tblk = 8192
    tpb = T // Tblk
    grid = B * tpb

    emb_hi = embedding.astype(jnp.bfloat16)
    emb_lo = (embedding - emb_hi.astype(jnp.float32)).astype(jnp.bfloat16)
    emb_cat = jnp.concatenate([emb_hi, emb_lo], axis=1)

    zq_st, idx3, counts2, loss2 = pl.pallas_call(
        functools.partial(_vq_block_kernel, num_codes=K),
        grid=(grid,),
        in_specs=[
            pl.BlockSpec((1, d, Tblk), lambda i: (i // tpb, 0, i % tpb)),
            pl.BlockSpec((K, d), lambda i: (0, 0)),
            pl.BlockSpec((K, 2 * d), lambda i: (0, 0)),
        ],
        out_specs=[
            pl.BlockSpec((1, d, Tblk), lambda i: (i // tpb, 0, i % tpb)),
            pl.BlockSpec((1, 1, Tblk), lambda i: (i, 0, 0)),
            pl.BlockSpec((1, K), lambda i: (0, 0)),
            pl.BlockSpec((1, 128), lambda i: (0, 0)),
        ],
        out_shape=[
            jax.ShapeDtypeStruct((B, d, T), jnp.float32),
            jax.ShapeDtypeStruct((grid, 1, Tblk), jnp.int32),
            jax.ShapeDtypeStruct((1, K), jnp.float32),
            jax.ShapeDtypeStruct((1, 128), jnp.float32),
        ],
    )(z_e, embedding, emb_cat)

    indices = idx3.reshape(B, T)
    counts = counts2.reshape(K)
    commit_loss = 0.25 * (loss2[0, 0] / (B * d * T))
    probs = counts / jnp.maximum(counts.sum(), 1.0)
    perplexity = jnp.exp(-jnp.sum(probs * jnp.log(probs + 1e-10)))
    return (zq_st, commit_loss, indices, perplexity, counts)
